# Initial kernel scaffold; baseline (speedup 1.0000x reference)
import jax
import jax.numpy as jnp
from jax.experimental import pallas as pl


def kernel(inputs, targets):
    raise NotImplementedError("write your pallas kernel here")


# speedup vs baseline: 19.6791x; 19.6791x over previous
import functools

import jax
import jax.numpy as jnp
from jax import lax
from jax.experimental import pallas as pl
from jax.experimental.pallas import tpu as pltpu
from jax.experimental.pallas import tpu_sc as plsc

_K = 18
_NB = 1 << _K
_HIST = 4 * _NB
_N = 16 * 512 * 512
_NC, _NS = 2, 16
_NW = _NC * _NS
_CHUNK = _N // _NW
_W = 2048
_NWIN = _CHUNK // _W
_VPW = _W // 16
_ROWS = _W // 128
_ZCHUNK = 4096
_TSLICE = _HIST // _NS


def _sc_hist_kernel(x_hbm, t_hbm, hist_hbm, max_hbm,
                    xv, tv, cidx, sidx, val, ones, zbuf, mbuf, hist_sh):
    c = lax.axis_index("c")
    s = lax.axis_index("s")
    wid = c * _NS + s

    def _fill(i, _):
        ones[pl.ds(i * 16, 16)] = jnp.full((16,), 1.0, jnp.float32)
        return 0
    lax.fori_loop(0, 8, _fill, 0)

    def _zfill(i, _):
        zbuf[pl.ds(i * 16, 16)] = jnp.zeros((16,), jnp.float32)
        return 0
    lax.fori_loop(0, _ZCHUNK // 16, _zfill, 0)

    def _zcopy(j, _):
        pltpu.sync_copy(zbuf, hist_sh.at[pl.ds(s * _TSLICE + j * _ZCHUNK,
                                               _ZCHUNK)])
        return 0
    lax.fori_loop(0, _TSLICE // _ZCHUNK, _zcopy, 0)
    plsc.subcore_barrier()

    def _window(w, maxe):
        base = wid * _CHUNK + w * _W
        pltpu.sync_copy(x_hbm.at[pl.ds(base, _W)], xv)
        pltpu.sync_copy(t_hbm.at[pl.ds(base, _W)], tv)

        def _vreg(v, mx):
            row = v // 8
            col = (v % 8) * 16
            x = xv[pl.ds(v * 16, 16)]
            t = tv[pl.ds(v * 16, 16)]
            e = jnp.abs(x - t.astype(jnp.float32))
            b = lax.bitcast_convert_type(e, jnp.int32)
            bucket = lax.shift_right_logical(b, 31 - _K)
            ci = t * _NB + bucket
            cidx[row, pl.ds(col, 16)] = ci
            sidx[row, pl.ds(col, 16)] = ci + 2 * _NB
            val[row, pl.ds(col, 16)] = e
            return jnp.maximum(mx, e)

        maxe = lax.fori_loop(0, _VPW, _vreg, maxe)

        def _scat(j, _):
            pltpu.sync_copy(ones, hist_sh.at[cidx.at[j]], add=True)
            pltpu.sync_copy(val.at[j], hist_sh.at[sidx.at[j]], add=True)
            return 0
        lax.fori_loop(0, _ROWS, _scat, 0)
        return maxe

    maxe = lax.fori_loop(0, _NWIN, _window, jnp.zeros((16,), jnp.float32))
    mbuf[...] = maxe
    pltpu.sync_copy(mbuf, max_hbm.at[wid])

    plsc.subcore_barrier()
    pltpu.sync_copy(hist_sh.at[pl.ds(s * _TSLICE, _TSLICE)],
                    hist_hbm.at[c, pl.ds(s * _TSLICE, _TSLICE)])


_NG = 16


def _block_suffix(x, g128, g128t):
    within = jax.lax.dot_general(x, g128, (((1,), (0,)), ((), ())),
                                 preferred_element_type=jnp.float32)
    rows = jnp.sum(x, axis=1, keepdims=True)
    tail_i = jax.lax.dot_general(g128t, rows, (((1,), (0,)), ((), ())),
                                 preferred_element_type=jnp.float32)
    return within + tail_i, jnp.sum(rows)


def _tc_reduce_kernel(hist_ref, max_ref, out_ref):
    h = hist_ref[...]
    cn = h[0, 0] + h[1, 0]
    cp = h[0, 1] + h[1, 1]
    sn = h[0, 2] + h[1, 2]
    sp = h[0, 3] + h[1, 3]
    t_total = jnp.sum(cp)

    r128 = lax.broadcasted_iota(jnp.int32, (128, 128), 0)
    c128 = lax.broadcasted_iota(jnp.int32, (128, 128), 1)
    g128 = (r128 > c128).astype(jnp.float32)
    g128t = (r128 < c128).astype(jnp.float32)

    na_blocks, pa_blocks, cn_tot, cp_tot = [], [], [], []
    for g in range(_NG):
        sl = slice(g * 128, (g + 1) * 128)
        na_b, cn_t = _block_suffix(cn[sl], g128, g128t)
        pa_b, cp_t = _block_suffix(cp[sl], g128, g128t)
        na_blocks.append(na_b)
        pa_blocks.append(pa_b)
        cn_tot.append(cn_t)
        cp_tot.append(cp_t)

    total = jnp.float32(0.0)
    cn_tail = jnp.float32(0.0)
    cp_tail = jnp.float32(0.0)
    for g in range(_NG - 1, -1, -1):
        sl = slice(g * 128, (g + 1) * 128)
        na = na_blocks[g] + cn_tail
        pa = pa_blocks[g] + cp_tail
        dn = t_total + na
        contrib = (sp[sl] / dn
                   + sn[sl] * (t_total - pa - cp[sl]) / (dn * (dn + cn[sl])))
        total = total + jnp.sum(contrib)
        cn_tail = cn_tail + cn_tot[g]
        cp_tail = cp_tail + cp_tot[g]

    emax = jnp.max(max_ref[...])
    res = jnp.where(t_total > 0, total, emax)
    out_ref[...] = jnp.broadcast_to(res, (1, 1))


def kernel(inputs, targets):
    if inputs.ndim == 4 and inputs.shape[1] == 1:
        inputs = jnp.squeeze(inputs, axis=1)
    x_flat = inputs.reshape(-1)
    t_flat = targets.reshape(-1)

    mesh = plsc.VectorSubcoreMesh(core_axis_name="c", subcore_axis_name="s")
    sc = functools.partial(
        pl.kernel,
        mesh=mesh,
        out_type=(
            jax.ShapeDtypeStruct((_NC, _HIST), jnp.float32),
            jax.ShapeDtypeStruct((_NW, 16), jnp.float32),
        ),
        scratch_types=[
            pltpu.VMEM((_W,), jnp.float32),
            pltpu.VMEM((_W,), jnp.int32),
            pltpu.VMEM((_ROWS, 128), jnp.int32),
            pltpu.VMEM((_ROWS, 128), jnp.int32),
            pltpu.VMEM((_ROWS, 128), jnp.float32),
            pltpu.VMEM((128,), jnp.float32),
            pltpu.VMEM((_ZCHUNK,), jnp.float32),
            pltpu.VMEM((16,), jnp.float32),
            pltpu.VMEM_SHARED((_HIST,), jnp.float32),
        ],
    )(_sc_hist_kernel)
    hist, maxes = sc(x_flat, t_flat)

    hist4 = hist.reshape(_NC, 4, 2048, 128)
    out = pl.pallas_call(
        _tc_reduce_kernel,
        out_shape=jax.ShapeDtypeStruct((1, 1), jnp.float32),
    )(hist4, maxes)
    return out[0, 0]


# speedup vs baseline: 35.1249x; 1.7849x over previous
import functools

import jax
import jax.numpy as jnp
from jax import lax
from jax.experimental import pallas as pl
from jax.experimental.pallas import tpu as pltpu
from jax.experimental.pallas import tpu_sc as plsc

_K = 18
_NB = 1 << _K
_HIST = 4 * _NB
_N = 16 * 512 * 512
_NC, _NS = 2, 16
_NW = _NC * _NS
_CHUNK = _N // _NW
_W = 8192
_NWIN = _CHUNK // _W
_VPW = _W // 16
_ROWS = _W // 128
_FIRE = 8
_ZCHUNK = 4096
_TSLICE = _HIST // _NS


def _sc_hist_kernel(x_hbm, t_hbm, hist_hbm, max_hbm,
                    xv, tv, cidx, sidx, val, ones, zbuf, mbuf, hist_sh, sem):
    c = lax.axis_index("c")
    s = lax.axis_index("s")
    wid = c * _NS + s

    def _fill(i, _):
        ones[pl.ds(i * 16, 16)] = jnp.full((16,), 1.0, jnp.float32)
        return 0
    lax.fori_loop(0, 8, _fill, 0)

    def _zfill(i, _):
        zbuf[pl.ds(i * 16, 16)] = jnp.zeros((16,), jnp.float32)
        return 0
    lax.fori_loop(0, _ZCHUNK // 16, _zfill, 0)

    def _zcopy(j, _):
        pltpu.sync_copy(zbuf, hist_sh.at[pl.ds(s * _TSLICE + j * _ZCHUNK,
                                               _ZCHUNK)])
        return 0
    lax.fori_loop(0, _TSLICE // _ZCHUNK, _zcopy, 0)
    plsc.subcore_barrier()

    def _window(w, maxe):
        base = wid * _CHUNK + w * _W
        pltpu.sync_copy(x_hbm.at[pl.ds(base, _W)], xv)
        pltpu.sync_copy(t_hbm.at[pl.ds(base, _W)], tv)

        def _vreg(v, mx):
            row = v // 8
            col = (v % 8) * 16
            x = xv[pl.ds(v * 16, 16)]
            t = tv[pl.ds(v * 16, 16)]
            e = jnp.abs(x - t.astype(jnp.float32))
            b = lax.bitcast_convert_type(e, jnp.int32)
            bucket = lax.shift_right_logical(b, 31 - _K)
            ci = t * _NB + bucket
            cidx[row, pl.ds(col, 16)] = ci
            sidx[row, pl.ds(col, 16)] = ci + 2 * _NB
            val[row, pl.ds(col, 16)] = e
            return jnp.maximum(mx, e)

        maxe = lax.fori_loop(0, _VPW, _vreg, maxe)

        for g in range(_ROWS // _FIRE):
            handles = []
            for k in range(_FIRE):
                j = g * _FIRE + k
                handles.append(pltpu.async_copy(
                    ones, hist_sh.at[cidx.at[j]], sem, add=True))
                handles.append(pltpu.async_copy(
                    val.at[j], hist_sh.at[sidx.at[j]], sem, add=True))
            for h in handles:
                h.wait()
        return maxe

    maxe = lax.fori_loop(0, _NWIN, _window, jnp.zeros((16,), jnp.float32))
    mbuf[...] = maxe
    pltpu.sync_copy(mbuf, max_hbm.at[wid])

    plsc.subcore_barrier()
    pltpu.sync_copy(hist_sh.at[pl.ds(s * _TSLICE, _TSLICE)],
                    hist_hbm.at[c, pl.ds(s * _TSLICE, _TSLICE)])


_NG = 16


def _block_suffix(x, g128, g128t):
    within = jax.lax.dot_general(x, g128, (((1,), (0,)), ((), ())),
                                 preferred_element_type=jnp.float32)
    rows = jnp.sum(x, axis=1, keepdims=True)
    tail_i = jax.lax.dot_general(g128t, rows, (((1,), (0,)), ((), ())),
                                 preferred_element_type=jnp.float32)
    return within + tail_i, jnp.sum(rows)


def _tc_reduce_kernel(hist_ref, max_ref, out_ref):
    h = hist_ref[...]
    cn = h[0, 0] + h[1, 0]
    cp = h[0, 1] + h[1, 1]
    sn = h[0, 2] + h[1, 2]
    sp = h[0, 3] + h[1, 3]
    t_total = jnp.sum(cp)

    r128 = lax.broadcasted_iota(jnp.int32, (128, 128), 0)
    c128 = lax.broadcasted_iota(jnp.int32, (128, 128), 1)
    g128 = (r128 > c128).astype(jnp.float32)
    g128t = (r128 < c128).astype(jnp.float32)

    na_blocks, pa_blocks, cn_tot, cp_tot = [], [], [], []
    for g in range(_NG):
        sl = slice(g * 128, (g + 1) * 128)
        na_b, cn_t = _block_suffix(cn[sl], g128, g128t)
        pa_b, cp_t = _block_suffix(cp[sl], g128, g128t)
        na_blocks.append(na_b)
        pa_blocks.append(pa_b)
        cn_tot.append(cn_t)
        cp_tot.append(cp_t)

    total = jnp.float32(0.0)
    cn_tail = jnp.float32(0.0)
    cp_tail = jnp.float32(0.0)
    for g in range(_NG - 1, -1, -1):
        sl = slice(g * 128, (g + 1) * 128)
        na = na_blocks[g] + cn_tail
        pa = pa_blocks[g] + cp_tail
        dn = t_total + na
        contrib = (sp[sl] / dn
                   + sn[sl] * (t_total - pa - cp[sl]) / (dn * (dn + cn[sl])))
        total = total + jnp.sum(contrib)
        cn_tail = cn_tail + cn_tot[g]
        cp_tail = cp_tail + cp_tot[g]

    emax = jnp.max(max_ref[...])
    res = jnp.where(t_total > 0, total, emax)
    out_ref[...] = jnp.broadcast_to(res, (1, 1))


def kernel(inputs, targets):
    if inputs.ndim == 4 and inputs.shape[1] == 1:
        inputs = jnp.squeeze(inputs, axis=1)
    x_flat = inputs.reshape(-1)
    t_flat = targets.reshape(-1)

    mesh = plsc.VectorSubcoreMesh(core_axis_name="c", subcore_axis_name="s")
    sc = functools.partial(
        pl.kernel,
        mesh=mesh,
        out_type=(
            jax.ShapeDtypeStruct((_NC, _HIST), jnp.float32),
            jax.ShapeDtypeStruct((_NW, 16), jnp.float32),
        ),
        scratch_types=[
            pltpu.VMEM((_W,), jnp.float32),
            pltpu.VMEM((_W,), jnp.int32),
            pltpu.VMEM((_ROWS, 128), jnp.int32),
            pltpu.VMEM((_ROWS, 128), jnp.int32),
            pltpu.VMEM((_ROWS, 128), jnp.float32),
            pltpu.VMEM((128,), jnp.float32),
            pltpu.VMEM((_ZCHUNK,), jnp.float32),
            pltpu.VMEM((16,), jnp.float32),
            pltpu.VMEM_SHARED((_HIST,), jnp.float32),
            pltpu.SemaphoreType.DMA,
        ],
    )(_sc_hist_kernel)
    hist, maxes = sc(x_flat, t_flat)

    hist4 = hist.reshape(_NC, 4, 2048, 128)
    out = pl.pallas_call(
        _tc_reduce_kernel,
        out_shape=jax.ShapeDtypeStruct((1, 1), jnp.float32),
    )(hist4, maxes)
    return out[0, 0]


# speedup vs baseline: 35.7442x; 1.0176x over previous
import functools

import jax
import jax.numpy as jnp
from jax import lax
from jax.experimental import pallas as pl
from jax.experimental.pallas import tpu as pltpu
from jax.experimental.pallas import tpu_sc as plsc

_K = 18
_NB = 1 << _K
_HIST = 4 * _NB
_N = 16 * 512 * 512
_NC, _NS = 2, 16
_NW = _NC * _NS
_CHUNK = _N // _NW
_W = 8192
_NWIN = _CHUNK // _W
_VPW = _W // 16
_ROWS = _W // 128
_FIRE = 16
_ZCHUNK = 4096
_TSLICE = _HIST // _NS


def _sc_hist_kernel(x_hbm, t_hbm, hist_hbm, max_hbm,
                    xv, tv, cidx, sidx, val, ones, zbuf, mbuf, hist_sh, sem):
    c = lax.axis_index("c")
    s = lax.axis_index("s")
    wid = c * _NS + s

    def _fill(i, _):
        ones[pl.ds(i * 16, 16)] = jnp.full((16,), 1.0, jnp.float32)
        return 0
    lax.fori_loop(0, 8, _fill, 0)

    def _zfill(i, _):
        zbuf[pl.ds(i * 16, 16)] = jnp.zeros((16,), jnp.float32)
        return 0
    lax.fori_loop(0, _ZCHUNK // 16, _zfill, 0)

    def _zcopy(j, _):
        pltpu.sync_copy(zbuf, hist_sh.at[pl.ds(s * _TSLICE + j * _ZCHUNK,
                                               _ZCHUNK)])
        return 0
    lax.fori_loop(0, _TSLICE // _ZCHUNK, _zcopy, 0)
    plsc.subcore_barrier()

    def _window(w, maxe):
        base = wid * _CHUNK + w * _W
        pltpu.sync_copy(x_hbm.at[pl.ds(base, _W)], xv)
        pltpu.sync_copy(t_hbm.at[pl.ds(base, _W)], tv)

        def _vreg(v, mx):
            row = v // 8
            col = (v % 8) * 16
            x = xv[pl.ds(v * 16, 16)]
            t = tv[pl.ds(v * 16, 16)]
            e = jnp.abs(x - t.astype(jnp.float32))
            b = lax.bitcast_convert_type(e, jnp.int32)
            bucket = lax.shift_right_logical(b, 31 - _K)
            ci = t * _NB + bucket
            cidx[row, pl.ds(col, 16)] = ci
            sidx[row, pl.ds(col, 16)] = ci + 2 * _NB
            val[row, pl.ds(col, 16)] = e
            return jnp.maximum(mx, e)

        maxe = lax.fori_loop(0, _VPW, _vreg, maxe)

        for g in range(_ROWS // _FIRE):
            handles = []
            for k in range(_FIRE):
                j = g * _FIRE + k
                handles.append(pltpu.async_copy(
                    ones, hist_sh.at[cidx.at[j]], sem, add=True))
                handles.append(pltpu.async_copy(
                    val.at[j], hist_sh.at[sidx.at[j]], sem, add=True))
            for h in handles:
                h.wait()
        return maxe

    maxe = lax.fori_loop(0, _NWIN, _window, jnp.zeros((16,), jnp.float32))
    mbuf[...] = maxe
    pltpu.sync_copy(mbuf, max_hbm.at[wid])

    plsc.subcore_barrier()
    pltpu.sync_copy(hist_sh.at[pl.ds(s * _TSLICE, _TSLICE)],
                    hist_hbm.at[c, pl.ds(s * _TSLICE, _TSLICE)])


_NG = 16


def _block_suffix(x, g128, g128t):
    within = jax.lax.dot_general(x, g128, (((1,), (0,)), ((), ())),
                                 preferred_element_type=jnp.float32)
    rows = jnp.sum(x, axis=1, keepdims=True)
    tail_i = jax.lax.dot_general(g128t, rows, (((1,), (0,)), ((), ())),
                                 preferred_element_type=jnp.float32)
    return within + tail_i, jnp.sum(rows)


def _tc_reduce_kernel(hist_ref, max_ref, out_ref):
    h = hist_ref[...]
    cn = h[0, 0] + h[1, 0]
    cp = h[0, 1] + h[1, 1]
    sn = h[0, 2] + h[1, 2]
    sp = h[0, 3] + h[1, 3]
    t_total = jnp.sum(cp)

    r128 = lax.broadcasted_iota(jnp.int32, (128, 128), 0)
    c128 = lax.broadcasted_iota(jnp.int32, (128, 128), 1)
    g128 = (r128 > c128).astype(jnp.float32)
    g128t = (r128 < c128).astype(jnp.float32)

    na_blocks, pa_blocks, cn_tot, cp_tot = [], [], [], []
    for g in range(_NG):
        sl = slice(g * 128, (g + 1) * 128)
        na_b, cn_t = _block_suffix(cn[sl], g128, g128t)
        pa_b, cp_t = _block_suffix(cp[sl], g128, g128t)
        na_blocks.append(na_b)
        pa_blocks.append(pa_b)
        cn_tot.append(cn_t)
        cp_tot.append(cp_t)

    total = jnp.float32(0.0)
    cn_tail = jnp.float32(0.0)
    cp_tail = jnp.float32(0.0)
    for g in range(_NG - 1, -1, -1):
        sl = slice(g * 128, (g + 1) * 128)
        na = na_blocks[g] + cn_tail
        pa = pa_blocks[g] + cp_tail
        dn = t_total + na
        contrib = (sp[sl] / dn
                   + sn[sl] * (t_total - pa - cp[sl]) / (dn * (dn + cn[sl])))
        total = total + jnp.sum(contrib)
        cn_tail = cn_tail + cn_tot[g]
        cp_tail = cp_tail + cp_tot[g]

    emax = jnp.max(max_ref[...])
    res = jnp.where(t_total > 0, total, emax)
    out_ref[...] = jnp.broadcast_to(res, (1, 1))


def kernel(inputs, targets):
    if inputs.ndim == 4 and inputs.shape[1] == 1:
        inputs = jnp.squeeze(inputs, axis=1)
    x_flat = inputs.reshape(-1)
    t_flat = targets.reshape(-1)

    mesh = plsc.VectorSubcoreMesh(core_axis_name="c", subcore_axis_name="s")
    sc = functools.partial(
        pl.kernel,
        mesh=mesh,
        out_type=(
            jax.ShapeDtypeStruct((_NC, _HIST), jnp.float32),
            jax.ShapeDtypeStruct((_NW, 16), jnp.float32),
        ),
        scratch_types=[
            pltpu.VMEM((_W,), jnp.float32),
            pltpu.VMEM((_W,), jnp.int32),
            pltpu.VMEM((_ROWS, 128), jnp.int32),
            pltpu.VMEM((_ROWS, 128), jnp.int32),
            pltpu.VMEM((_ROWS, 128), jnp.float32),
            pltpu.VMEM((128,), jnp.float32),
            pltpu.VMEM((_ZCHUNK,), jnp.float32),
            pltpu.VMEM((16,), jnp.float32),
            pltpu.VMEM_SHARED((_HIST,), jnp.float32),
            pltpu.SemaphoreType.DMA,
        ],
    )(_sc_hist_kernel)
    hist, maxes = sc(x_flat, t_flat)

    hist4 = hist.reshape(_NC, 4, 2048, 128)
    out = pl.pallas_call(
        _tc_reduce_kernel,
        out_shape=jax.ShapeDtypeStruct((1, 1), jnp.float32),
    )(hist4, maxes)
    return out[0, 0]
